# baseline (speedup 1.0000x reference)
import jax
import jax.numpy as jnp
from jax.experimental import pallas as pl
from jax.experimental.pallas import tpu as pltpu

_BM = 200


def _body(x_ref, w_ref, adj_ref, b_ref, o_ref, s_ref):
    m = pl.program_id(0)

    @pl.when(m == 0)
    def _support():
        s_ref[...] = jnp.dot(
            x_ref[...].astype(jnp.bfloat16),
            w_ref[...].astype(jnp.bfloat16),
            preferred_element_type=jnp.float32,
        ).astype(jnp.bfloat16)

    @pl.when(m > 0)
    def _aggregate():
        o_ref[...] = (
            jnp.dot(
                adj_ref[...].astype(jnp.bfloat16),
                s_ref[...],
                preferred_element_type=jnp.float32,
            )
            + b_ref[...]
        )


def kernel(x, adj_norm, weight, bias):
    n, d_in = x.shape
    d_out = weight.shape[1]

    return pl.pallas_call(
        _body,
        grid=(1 + n // _BM,),
        in_specs=[
            pl.BlockSpec((n, d_in), lambda m: (0, 0)),
            pl.BlockSpec((d_in, d_out), lambda m: (0, 0)),
            pl.BlockSpec((_BM, n), lambda m: (jnp.maximum(m - 1, 0), 0)),
            pl.BlockSpec((1, d_out), lambda m: (0, 0)),
        ],
        out_specs=pl.BlockSpec((_BM, d_out), lambda m: (jnp.maximum(m - 1, 0), 0)),
        out_shape=jax.ShapeDtypeStruct((n, d_out), jnp.float32),
        scratch_shapes=[pltpu.VMEM((n, d_out), jnp.bfloat16)],
        compiler_params=pltpu.CompilerParams(
            dimension_semantics=("arbitrary",),
        ),
    )(x, weight, adj_norm, bias.reshape(1, d_out))


# speedup vs baseline: 1.0059x; 1.0059x over previous
import jax
import jax.numpy as jnp
from jax.experimental import pallas as pl
from jax.experimental.pallas import tpu as pltpu

_BM = 400


def _body(x_ref, w_ref, adj_ref, b_ref, o_ref, s_ref):
    m = pl.program_id(0)

    @pl.when(m == 0)
    def _support():
        s_ref[...] = jnp.dot(
            x_ref[...].astype(jnp.bfloat16),
            w_ref[...].astype(jnp.bfloat16),
            preferred_element_type=jnp.float32,
        ).astype(jnp.bfloat16)

    @pl.when(m > 0)
    def _aggregate():
        o_ref[...] = (
            jnp.dot(
                adj_ref[...].astype(jnp.bfloat16),
                s_ref[...],
                preferred_element_type=jnp.float32,
            )
            + b_ref[...]
        )


def kernel(x, adj_norm, weight, bias):
    n, d_in = x.shape
    d_out = weight.shape[1]

    return pl.pallas_call(
        _body,
        grid=(1 + n // _BM,),
        in_specs=[
            pl.BlockSpec((n, d_in), lambda m: (0, 0)),
            pl.BlockSpec((d_in, d_out), lambda m: (0, 0)),
            pl.BlockSpec((_BM, n), lambda m: (jnp.maximum(m - 1, 0), 0)),
            pl.BlockSpec((1, d_out), lambda m: (0, 0)),
        ],
        out_specs=pl.BlockSpec((_BM, d_out), lambda m: (jnp.maximum(m - 1, 0), 0)),
        out_shape=jax.ShapeDtypeStruct((n, d_out), jnp.float32),
        scratch_shapes=[pltpu.VMEM((n, d_out), jnp.bfloat16)],
        compiler_params=pltpu.CompilerParams(
            dimension_semantics=("arbitrary",),
        ),
    )(x, weight, adj_norm, bias.reshape(1, d_out))


# speedup vs baseline: 1.0095x; 1.0036x over previous
import jax
import jax.numpy as jnp
from jax.experimental import pallas as pl
from jax.experimental.pallas import tpu as pltpu

_BM = 400


def _body(x_ref, w_ref, adj_ref, b_ref, o_ref, s_ref):
    m = pl.program_id(0)

    @pl.when(m == 0)
    def _support():
        s_ref[...] = jnp.dot(
            x_ref[...],
            w_ref[...],
            precision=jax.lax.Precision.DEFAULT,
            preferred_element_type=jnp.float32,
        )

    @pl.when(m > 0)
    def _aggregate():
        o_ref[...] = (
            jnp.dot(
                adj_ref[...],
                s_ref[...],
                precision=jax.lax.Precision.DEFAULT,
                preferred_element_type=jnp.float32,
            )
            + b_ref[...]
        )


def kernel(x, adj_norm, weight, bias):
    n, d_in = x.shape
    d_out = weight.shape[1]

    return pl.pallas_call(
        _body,
        grid=(1 + n // _BM,),
        in_specs=[
            pl.BlockSpec((n, d_in), lambda m: (0, 0)),
            pl.BlockSpec((d_in, d_out), lambda m: (0, 0)),
            pl.BlockSpec((_BM, n), lambda m: (jnp.maximum(m - 1, 0), 0)),
            pl.BlockSpec((1, d_out), lambda m: (0, 0)),
        ],
        out_specs=pl.BlockSpec((_BM, d_out), lambda m: (jnp.maximum(m - 1, 0), 0)),
        out_shape=jax.ShapeDtypeStruct((n, d_out), jnp.float32),
        scratch_shapes=[pltpu.VMEM((n, d_out), jnp.float32)],
        compiler_params=pltpu.CompilerParams(
            dimension_semantics=("arbitrary",),
        ),
    )(x, weight, adj_norm, bias.reshape(1, d_out))
